# baseline (speedup 1.0000x reference)
import functools

import jax
import jax.numpy as jnp
from jax import lax
from jax.experimental import pallas as pl
from jax.experimental.pallas import tpu as pltpu
from jax.experimental.pallas import tpu_sc as plsc

NUM_CORES = 2
NUM_SUBCORES = 16
NUM_WORKERS = NUM_CORES * NUM_SUBCORES
LANES = 16

VOCAB = 1000000
EMB_DIM = 512
BATCH = 16384

PAIRS_PER_WORKER = BATCH // NUM_WORKERS
CHUNK_PAIRS = 32
CHUNK_ROWS = 2 * CHUNK_PAIRS
N_CHUNKS = PAIRS_PER_WORKER // CHUNK_PAIRS
NBUF = 2
PAIR_GROUP = 8
N_SLICES = EMB_DIM // LANES


def _sc_body(table, idx2, w3, b3p, out,
             idx2_v, idxflat_v, w3_v, wa0_v, wa1_v, wb0_v, wb1_v,
             x0_v, x1_v, out_v, b3_v, buf0, buf1, sem0, sem1):
  wid = lax.axis_index("s") * NUM_CORES + lax.axis_index("c")
  pair_base = wid * PAIRS_PER_WORKER

  pltpu.sync_copy(idx2.at[pl.ds(pair_base, PAIRS_PER_WORKER), :], idx2_v)
  pltpu.sync_copy(w3, w3_v)
  pltpu.sync_copy(b3p, b3_v)

  iota = lax.iota(jnp.int32, 16)
  col0 = jnp.zeros((LANES,), jnp.int32)
  col1 = col0 + 1

  def deint(j, _):
    f = 16 * j + iota
    r = lax.shift_right_logical(f, 1)
    c = lax.bitwise_and(f, 1)
    idxflat_v[pl.ds(16 * j, 16)] = plsc.load_gather(idx2_v, [r, c])
    return 0

  lax.fori_loop(0, (2 * PAIRS_PER_WORKER) // 16, deint, 0)

  def w_prep(j, _):
    d = 16 * j + iota
    wd0 = plsc.load_gather(w3_v, [d, col0])
    ws0 = plsc.load_gather(w3_v, [d + 512, col0])
    wa0 = plsc.load_gather(w3_v, [d + 1024, col0])
    wb0 = plsc.load_gather(w3_v, [d + 1536, col0])
    wd1 = plsc.load_gather(w3_v, [d, col1])
    ws1 = plsc.load_gather(w3_v, [d + 512, col1])
    wa1 = plsc.load_gather(w3_v, [d + 1024, col1])
    wb1 = plsc.load_gather(w3_v, [d + 1536, col1])
    sl = pl.ds(16 * j, 16)
    wa0_v[sl] = wd0 + ws0 + wa0
    wa1_v[sl] = wd1 + ws1 + wa1
    wb0_v[sl] = ws0 - wd0 + wb0
    wb1_v[sl] = ws1 - wd1 + wb1
    return 0

  lax.fori_loop(0, N_SLICES, w_prep, 0)

  bufs = (buf0, buf1)
  sems = (sem0, sem1)

  def gather_chunk(cc, b):
    idx_sl = idxflat_v.at[pl.ds(cc * CHUNK_ROWS, CHUNK_ROWS)]
    pltpu.make_async_copy(table.at[idx_sl], bufs[b], sems[b]).start()

  def wait_chunk(cc, b):
    idx_sl = idxflat_v.at[pl.ds(cc * CHUNK_ROWS, CHUNK_ROWS)]
    pltpu.make_async_copy(table.at[idx_sl], bufs[b], sems[b]).wait()

  for b in range(NBUF):
    gather_chunk(jnp.int32(b), b)

  zero = jnp.zeros((LANES,), jnp.float32)
  last = iota == 15

  def compute_chunk(cc, buf):
    def group_body(g, _):
      i0 = g * PAIR_GROUP
      acc0 = [zero] * PAIR_GROUP
      acc1 = [zero] * PAIR_GROUP
      for j in range(N_SLICES):
        sl = pl.ds(16 * j, 16)
        w0 = wa0_v[sl]
        w1 = wa1_v[sl]
        for q in range(PAIR_GROUP):
          va = buf[(i0 + q) * 2, sl]
          acc0[q] = acc0[q] + va * w0
          acc1[q] = acc1[q] + va * w1
      for j in range(N_SLICES):
        sl = pl.ds(16 * j, 16)
        w0 = wb0_v[sl]
        w1 = wb1_v[sl]
        for q in range(PAIR_GROUP):
          vb = buf[(i0 + q) * 2 + 1, sl]
          acc0[q] = acc0[q] + vb * w0
          acc1[q] = acc1[q] + vb * w1
      for q in range(PAIR_GROUP):
        p = jnp.broadcast_to(cc * CHUNK_PAIRS + i0 + q, (16,))
        plsc.store_scatter(x0_v, [p], plsc.cumsum(acc0[q]), mask=last)
        plsc.store_scatter(x1_v, [p], plsc.cumsum(acc1[q]), mask=last)
      return 0

    lax.fori_loop(0, CHUNK_PAIRS // PAIR_GROUP, group_body, 0)

  def ring_body(g, _):
    for b in range(NBUF):
      cc = g * NBUF + b
      wait_chunk(cc, b)
      compute_chunk(cc, bufs[b])

      @pl.when(cc + NBUF < N_CHUNKS)
      def _():
        gather_chunk(cc + NBUF, b)

    return 0

  lax.fori_loop(0, N_CHUNKS // NBUF, ring_body, 0)

  bv = b3_v[pl.ds(0, 16)]
  b0 = bv[0]
  b1 = bv[1]

  def epilogue(j, _):
    sl = pl.ds(16 * j, 16)
    x0 = x0_v[sl] + b0
    x1 = x1_v[sl] + b1
    m = jnp.maximum(x0, x1)
    t = jnp.exp(-jnp.abs(x0 - x1))
    z = t / (2.0 + t)
    z2 = z * z
    log1p_t = 2.0 * z * (1.0 + z2 * (1.0 / 3.0 + z2 * (0.2 + z2 * (1.0 / 7.0 + z2 * (1.0 / 9.0)))))
    lse = m + log1p_t
    ids = 16 * j + iota
    plsc.store_scatter(out_v, [ids, col0], x0 - lse)
    plsc.store_scatter(out_v, [ids, col1], x1 - lse)
    return 0

  lax.fori_loop(0, PAIRS_PER_WORKER // 16, epilogue, 0)

  pltpu.sync_copy(out_v, out.at[pl.ds(pair_base, PAIRS_PER_WORKER), :])


@functools.partial(
    pl.kernel,
    out_type=jax.ShapeDtypeStruct((BATCH, 2), jnp.float32),
    mesh=plsc.VectorSubcoreMesh(core_axis_name="c", subcore_axis_name="s"),
    scratch_types=[
        pltpu.VMEM((PAIRS_PER_WORKER, 2), jnp.int32),
        pltpu.VMEM((2 * PAIRS_PER_WORKER,), jnp.int32),
        pltpu.VMEM((2048, 2), jnp.float32),
        pltpu.VMEM((EMB_DIM,), jnp.float32),
        pltpu.VMEM((EMB_DIM,), jnp.float32),
        pltpu.VMEM((EMB_DIM,), jnp.float32),
        pltpu.VMEM((EMB_DIM,), jnp.float32),
        pltpu.VMEM((PAIRS_PER_WORKER,), jnp.float32),
        pltpu.VMEM((PAIRS_PER_WORKER,), jnp.float32),
        pltpu.VMEM((PAIRS_PER_WORKER, 2), jnp.float32),
        pltpu.VMEM((LANES,), jnp.float32),
        pltpu.VMEM((CHUNK_ROWS, EMB_DIM), jnp.float32),
        pltpu.VMEM((CHUNK_ROWS, EMB_DIM), jnp.float32),
        pltpu.SemaphoreType.DMA,
        pltpu.SemaphoreType.DMA,
    ],
    compiler_params=pltpu.CompilerParams(needs_layout_passes=False,
                                         use_tc_tiling_on_sc=False),
)
def _siamese_sc(table, idx2, w3, b3p, out, *scratch):
  _sc_body(table, idx2, w3, b3p, out, *scratch)


def kernel(inputs, epoch, table, W3, b3):
  del epoch
  b3_pad = jnp.pad(b3, (0, LANES - b3.shape[0]))
  return _siamese_sc(table, inputs, W3, b3_pad)


# speedup vs baseline: 21.5321x; 21.5321x over previous
import functools

import jax
import jax.numpy as jnp
from jax import lax
from jax.experimental import pallas as pl
from jax.experimental.pallas import tpu as pltpu
from jax.experimental.pallas import tpu_sc as plsc

NUM_CORES = 2
NUM_SUBCORES = 16
NUM_WORKERS = NUM_CORES * NUM_SUBCORES
LANES = 16

VOCAB = 1000000
EMB_DIM = 512
BATCH = 16384

PAIRS_PER_WORKER = BATCH // NUM_WORKERS
CHUNK_PAIRS = 32
N_CHUNKS = PAIRS_PER_WORKER // CHUNK_PAIRS
NBUF = 2
PAIR_GROUP = 8
N_SLICES = EMB_DIM // LANES


def _sc_body(table, idxa, idxb, w3f, b3p, out0, out1,
             idxa_v, idxb_v, w3_v, wa0_v, wa1_v, wb0_v, wb1_v,
             x0_v, x1_v, b3_v, buf0, buf1, sem0, sem1):
  wid = lax.axis_index("s") * NUM_CORES + lax.axis_index("c")
  base = wid * PAIRS_PER_WORKER

  pltpu.sync_copy(idxa.at[pl.ds(base, PAIRS_PER_WORKER)], idxa_v)
  pltpu.sync_copy(idxb.at[pl.ds(base, PAIRS_PER_WORKER)], idxb_v)
  pltpu.sync_copy(w3f, w3_v)
  pltpu.sync_copy(b3p, b3_v)

  def w_prep(j, _):
    d2 = 2 * (16 * j + lax.iota(jnp.int32, 16))
    wd0 = plsc.load_gather(w3_v, [d2])
    ws0 = plsc.load_gather(w3_v, [d2 + 1024])
    wa0 = plsc.load_gather(w3_v, [d2 + 2048])
    wb0 = plsc.load_gather(w3_v, [d2 + 3072])
    wd1 = plsc.load_gather(w3_v, [d2 + 1])
    ws1 = plsc.load_gather(w3_v, [d2 + 1025])
    wa1 = plsc.load_gather(w3_v, [d2 + 2049])
    wb1 = plsc.load_gather(w3_v, [d2 + 3073])
    sl = pl.ds(16 * j, 16)
    wa0_v[sl] = wd0 + ws0 + wa0
    wa1_v[sl] = wd1 + ws1 + wa1
    wb0_v[sl] = ws0 - wd0 + wb0
    wb1_v[sl] = ws1 - wd1 + wb1
    return 0

  lax.fori_loop(0, N_SLICES, w_prep, 0)

  bufs = (buf0, buf1)
  sems = (sem0, sem1)

  def gather_chunk(cc, b):
    a_sl = idxa_v.at[pl.ds(cc * CHUNK_PAIRS, CHUNK_PAIRS)]
    b_sl = idxb_v.at[pl.ds(cc * CHUNK_PAIRS, CHUNK_PAIRS)]
    pltpu.make_async_copy(table.at[a_sl],
                          bufs[b].at[pl.ds(0, CHUNK_PAIRS)], sems[b]).start()
    pltpu.make_async_copy(table.at[b_sl],
                          bufs[b].at[pl.ds(CHUNK_PAIRS, CHUNK_PAIRS)],
                          sems[b]).start()

  def wait_chunk(cc, b):
    a_sl = idxa_v.at[pl.ds(cc * CHUNK_PAIRS, CHUNK_PAIRS)]
    b_sl = idxb_v.at[pl.ds(cc * CHUNK_PAIRS, CHUNK_PAIRS)]
    pltpu.make_async_copy(table.at[a_sl],
                          bufs[b].at[pl.ds(0, CHUNK_PAIRS)], sems[b]).wait()
    pltpu.make_async_copy(table.at[b_sl],
                          bufs[b].at[pl.ds(CHUNK_PAIRS, CHUNK_PAIRS)],
                          sems[b]).wait()

  for b in range(NBUF):
    gather_chunk(jnp.int32(b), b)

  zero = jnp.zeros((LANES,), jnp.float32)
  last = lax.iota(jnp.int32, 16) == 15

  def compute_chunk(cc, buf):
    def group_body(g, _):
      i0 = g * PAIR_GROUP
      acc0 = [zero] * PAIR_GROUP
      acc1 = [zero] * PAIR_GROUP
      for j in range(N_SLICES):
        sl = pl.ds(16 * j, 16)
        w0 = wa0_v[sl]
        w1 = wa1_v[sl]
        for q in range(PAIR_GROUP):
          va = buf[i0 + q, sl]
          acc0[q] = acc0[q] + va * w0
          acc1[q] = acc1[q] + va * w1
      for j in range(N_SLICES):
        sl = pl.ds(16 * j, 16)
        w0 = wb0_v[sl]
        w1 = wb1_v[sl]
        for q in range(PAIR_GROUP):
          vb = buf[CHUNK_PAIRS + i0 + q, sl]
          acc0[q] = acc0[q] + vb * w0
          acc1[q] = acc1[q] + vb * w1
      for q in range(PAIR_GROUP):
        p = jnp.broadcast_to(cc * CHUNK_PAIRS + i0 + q, (16,))
        plsc.store_scatter(x0_v, [p], plsc.cumsum(acc0[q]), mask=last)
        plsc.store_scatter(x1_v, [p], plsc.cumsum(acc1[q]), mask=last)
      return 0

    lax.fori_loop(0, CHUNK_PAIRS // PAIR_GROUP, group_body, 0)

  def ring_body(g, _):
    for b in range(NBUF):
      cc = g * NBUF + b
      wait_chunk(cc, b)
      compute_chunk(cc, bufs[b])

      @pl.when(cc + NBUF < N_CHUNKS)
      def _():
        gather_chunk(cc + NBUF, b)

    return 0

  lax.fori_loop(0, N_CHUNKS // NBUF, ring_body, 0)

  bv = b3_v[pl.ds(0, 16)]
  b0 = bv[0]
  b1 = bv[1]

  def epilogue(j, _):
    sl = pl.ds(16 * j, 16)
    x0 = x0_v[sl] + b0
    x1 = x1_v[sl] + b1
    m = jnp.maximum(x0, x1)
    t = jnp.exp(-jnp.abs(x0 - x1))
    z = t / (2.0 + t)
    z2 = z * z
    log1p_t = 2.0 * z * (1.0 + z2 * (1.0 / 3.0 + z2 * (0.2 + z2 * (1.0 / 7.0 + z2 * (1.0 / 9.0)))))
    lse = m + log1p_t
    x0_v[sl] = x0 - lse
    x1_v[sl] = x1 - lse
    return 0

  lax.fori_loop(0, PAIRS_PER_WORKER // 16, epilogue, 0)

  pltpu.sync_copy(x0_v, out0.at[pl.ds(base, PAIRS_PER_WORKER)])
  pltpu.sync_copy(x1_v, out1.at[pl.ds(base, PAIRS_PER_WORKER)])


@functools.partial(
    pl.kernel,
    out_type=(jax.ShapeDtypeStruct((BATCH,), jnp.float32),
              jax.ShapeDtypeStruct((BATCH,), jnp.float32)),
    mesh=plsc.VectorSubcoreMesh(core_axis_name="c", subcore_axis_name="s"),
    scratch_types=[
        pltpu.VMEM((PAIRS_PER_WORKER,), jnp.int32),
        pltpu.VMEM((PAIRS_PER_WORKER,), jnp.int32),
        pltpu.VMEM((4096,), jnp.float32),
        pltpu.VMEM((EMB_DIM,), jnp.float32),
        pltpu.VMEM((EMB_DIM,), jnp.float32),
        pltpu.VMEM((EMB_DIM,), jnp.float32),
        pltpu.VMEM((EMB_DIM,), jnp.float32),
        pltpu.VMEM((PAIRS_PER_WORKER,), jnp.float32),
        pltpu.VMEM((PAIRS_PER_WORKER,), jnp.float32),
        pltpu.VMEM((LANES,), jnp.float32),
        pltpu.VMEM((2 * CHUNK_PAIRS, EMB_DIM), jnp.float32),
        pltpu.VMEM((2 * CHUNK_PAIRS, EMB_DIM), jnp.float32),
        pltpu.SemaphoreType.DMA,
        pltpu.SemaphoreType.DMA,
    ],
    compiler_params=pltpu.CompilerParams(needs_layout_passes=False),
)
def _siamese_sc(table, idxa, idxb, w3f, b3p, out0, out1, *scratch):
  _sc_body(table, idxa, idxb, w3f, b3p, out0, out1, *scratch)


def kernel(inputs, epoch, table, W3, b3):
  del epoch
  idx_a = inputs[:, 0]
  idx_b = inputs[:, 1]
  w3_flat = W3.reshape(-1)
  b3_pad = jnp.pad(b3, (0, LANES - b3.shape[0]))
  out0, out1 = _siamese_sc(table, idx_a, idx_b, w3_flat, b3_pad)
  return jnp.stack([out0, out1], axis=1)


# speedup vs baseline: 22.2847x; 1.0350x over previous
import functools

import jax
import jax.numpy as jnp
from jax import lax
from jax.experimental import pallas as pl
from jax.experimental.pallas import tpu as pltpu
from jax.experimental.pallas import tpu_sc as plsc

NUM_CORES = 2
NUM_SUBCORES = 16
NUM_WORKERS = NUM_CORES * NUM_SUBCORES
LANES = 16

VOCAB = 1000000
EMB_DIM = 512
BATCH = 16384

PAIRS_PER_WORKER = BATCH // NUM_WORKERS
CHUNK_PAIRS = 32
N_CHUNKS = PAIRS_PER_WORKER // CHUNK_PAIRS
NBUF = 2
PAIR_GROUP = 8
N_SLICES = EMB_DIM // LANES


def _sc_body(table, idxa, idxb, w3b, out0, out1,
             idxa_v, idxb_v, w3_v, wa0_v, wa1_v, wb0_v, wb1_v,
             x0_v, x1_v, buf0, buf1, sem0, sem1):
  wid = lax.axis_index("s") * NUM_CORES + lax.axis_index("c")
  base = wid * PAIRS_PER_WORKER

  pltpu.sync_copy(idxa.at[pl.ds(base, PAIRS_PER_WORKER)], idxa_v)
  pltpu.sync_copy(idxb.at[pl.ds(base, PAIRS_PER_WORKER)], idxb_v)

  bufs = (buf0, buf1)
  sems = (sem0, sem1)

  def gather_chunk(cc, b):
    a_sl = idxa_v.at[pl.ds(cc * CHUNK_PAIRS, CHUNK_PAIRS)]
    b_sl = idxb_v.at[pl.ds(cc * CHUNK_PAIRS, CHUNK_PAIRS)]
    pltpu.make_async_copy(table.at[a_sl],
                          bufs[b].at[pl.ds(0, CHUNK_PAIRS)], sems[b]).start()
    pltpu.make_async_copy(table.at[b_sl],
                          bufs[b].at[pl.ds(CHUNK_PAIRS, CHUNK_PAIRS)],
                          sems[b]).start()

  def wait_chunk(cc, b):
    a_sl = idxa_v.at[pl.ds(cc * CHUNK_PAIRS, CHUNK_PAIRS)]
    b_sl = idxb_v.at[pl.ds(cc * CHUNK_PAIRS, CHUNK_PAIRS)]
    pltpu.make_async_copy(table.at[a_sl],
                          bufs[b].at[pl.ds(0, CHUNK_PAIRS)], sems[b]).wait()
    pltpu.make_async_copy(table.at[b_sl],
                          bufs[b].at[pl.ds(CHUNK_PAIRS, CHUNK_PAIRS)],
                          sems[b]).wait()

  for b in range(NBUF):
    gather_chunk(jnp.int32(b), b)

  pltpu.sync_copy(w3b, w3_v)

  def w_prep(j, _):
    d2 = 2 * (16 * j + lax.iota(jnp.int32, 16))
    wd0 = plsc.load_gather(w3_v, [d2])
    ws0 = plsc.load_gather(w3_v, [d2 + 1024])
    wa0 = plsc.load_gather(w3_v, [d2 + 2048])
    wb0 = plsc.load_gather(w3_v, [d2 + 3072])
    wd1 = plsc.load_gather(w3_v, [d2 + 1])
    ws1 = plsc.load_gather(w3_v, [d2 + 1025])
    wa1 = plsc.load_gather(w3_v, [d2 + 2049])
    wb1 = plsc.load_gather(w3_v, [d2 + 3073])
    sl = pl.ds(16 * j, 16)
    wa0_v[sl] = wd0 + ws0 + wa0
    wa1_v[sl] = wd1 + ws1 + wa1
    wb0_v[sl] = ws0 - wd0 + wb0
    wb1_v[sl] = ws1 - wd1 + wb1
    return 0

  lax.fori_loop(0, N_SLICES, w_prep, 0)

  zero = jnp.zeros((LANES,), jnp.float32)
  last = lax.iota(jnp.int32, 16) == 15

  def compute_chunk(cc, buf):
    def group_body(g, _):
      i0 = g * PAIR_GROUP
      acc0 = [zero] * PAIR_GROUP
      acc1 = [zero] * PAIR_GROUP
      for j in range(N_SLICES):
        sl = pl.ds(16 * j, 16)
        w0 = wa0_v[sl]
        w1 = wa1_v[sl]
        for q in range(PAIR_GROUP):
          va = buf[i0 + q, sl]
          acc0[q] = acc0[q] + va * w0
          acc1[q] = acc1[q] + va * w1
      for j in range(N_SLICES):
        sl = pl.ds(16 * j, 16)
        w0 = wb0_v[sl]
        w1 = wb1_v[sl]
        for q in range(PAIR_GROUP):
          vb = buf[CHUNK_PAIRS + i0 + q, sl]
          acc0[q] = acc0[q] + vb * w0
          acc1[q] = acc1[q] + vb * w1
      for q in range(PAIR_GROUP):
        p = jnp.broadcast_to(cc * CHUNK_PAIRS + i0 + q, (16,))
        plsc.store_scatter(x0_v, [p], plsc.cumsum(acc0[q]), mask=last)
        plsc.store_scatter(x1_v, [p], plsc.cumsum(acc1[q]), mask=last)
      return 0

    lax.fori_loop(0, CHUNK_PAIRS // PAIR_GROUP, group_body, 0)

  def ring_body(g, _):
    for b in range(NBUF):
      cc = g * NBUF + b
      wait_chunk(cc, b)
      compute_chunk(cc, bufs[b])

      @pl.when(cc + NBUF < N_CHUNKS)
      def _():
        gather_chunk(cc + NBUF, b)

    return 0

  lax.fori_loop(0, N_CHUNKS // NBUF, ring_body, 0)

  bv = w3_v[pl.ds(4096, 16)]
  b0 = bv[0]
  b1 = bv[1]

  def epilogue(j, _):
    sl = pl.ds(16 * j, 16)
    x0 = x0_v[sl] + b0
    x1 = x1_v[sl] + b1
    m = jnp.maximum(x0, x1)
    t = jnp.exp(-jnp.abs(x0 - x1))
    z = t / (2.0 + t)
    z2 = z * z
    log1p_t = 2.0 * z * (1.0 + z2 * (1.0 / 3.0 + z2 * (0.2 + z2 * (1.0 / 7.0 + z2 * (1.0 / 9.0)))))
    lse = m + log1p_t
    x0_v[sl] = x0 - lse
    x1_v[sl] = x1 - lse
    return 0

  lax.fori_loop(0, PAIRS_PER_WORKER // 16, epilogue, 0)

  pltpu.sync_copy(x0_v, out0.at[pl.ds(base, PAIRS_PER_WORKER)])
  pltpu.sync_copy(x1_v, out1.at[pl.ds(base, PAIRS_PER_WORKER)])


@functools.partial(
    pl.kernel,
    out_type=(jax.ShapeDtypeStruct((BATCH,), jnp.float32),
              jax.ShapeDtypeStruct((BATCH,), jnp.float32)),
    mesh=plsc.VectorSubcoreMesh(core_axis_name="c", subcore_axis_name="s"),
    scratch_types=[
        pltpu.VMEM((PAIRS_PER_WORKER,), jnp.int32),
        pltpu.VMEM((PAIRS_PER_WORKER,), jnp.int32),
        pltpu.VMEM((4112,), jnp.float32),
        pltpu.VMEM((EMB_DIM,), jnp.float32),
        pltpu.VMEM((EMB_DIM,), jnp.float32),
        pltpu.VMEM((EMB_DIM,), jnp.float32),
        pltpu.VMEM((EMB_DIM,), jnp.float32),
        pltpu.VMEM((PAIRS_PER_WORKER,), jnp.float32),
        pltpu.VMEM((PAIRS_PER_WORKER,), jnp.float32),
        pltpu.VMEM((2 * CHUNK_PAIRS, EMB_DIM), jnp.float32),
        pltpu.VMEM((2 * CHUNK_PAIRS, EMB_DIM), jnp.float32),
        pltpu.SemaphoreType.DMA,
        pltpu.SemaphoreType.DMA,
    ],
    compiler_params=pltpu.CompilerParams(needs_layout_passes=False),
)
def _siamese_sc(table, idxa, idxb, w3b, out0, out1, *scratch):
  _sc_body(table, idxa, idxb, w3b, out0, out1, *scratch)


def kernel(inputs, epoch, table, W3, b3):
  del epoch
  idx_a = inputs[:, 0]
  idx_b = inputs[:, 1]
  w3b = jnp.concatenate([W3.reshape(-1), b3,
                         jnp.zeros((LANES - b3.shape[0],), jnp.float32)])
  out0, out1 = _siamese_sc(table, idx_a, idx_b, w3b)
  return jnp.stack([out0, out1], axis=1)
